# Initial kernel scaffold; baseline (speedup 1.0000x reference)
import jax
import jax.numpy as jnp
from jax.experimental import pallas as pl


def kernel(x, edge_index, batch, conv_W, conv_b, bn1_g, bn1_b, lin1_W, lin1_b, bn2_g, bn2_b, lin2_W, lin2_b, bn3_g, bn3_b):
    raise NotImplementedError("write your pallas kernel here")


# speedup vs baseline: 10.1976x; 10.1976x over previous
import functools

import jax
import jax.numpy as jnp
from jax import lax
from jax.experimental import pallas as pl
from jax.experimental.pallas import tpu as pltpu
from jax.experimental.pallas import tpu_sc as plsc

_NC = 2
_NS = 16
_NW = _NC * _NS
_C = 128
_DEGW = 128
_IB = 16


def _sc_degree(dst2, *, chunks, acc_rows):
    rows_per_tile = acc_rows // _NS
    mesh = plsc.VectorSubcoreMesh(core_axis_name="c", subcore_axis_name="s")

    @functools.partial(
        pl.kernel,
        out_type=jax.ShapeDtypeStruct((_NC * acc_rows, _DEGW), jnp.float32),
        mesh=mesh,
        scratch_types=[
            pltpu.VMEM((chunks, _C), jnp.int32),
            pltpu.VMEM((_C, _DEGW), jnp.float32),
            pltpu.VMEM_SHARED((acc_rows, _DEGW), jnp.float32),
        ],
    )
    def k(dst_hbm, out_hbm, idx_v, ones_v, deg_sp):
        c = lax.axis_index("c")
        s = lax.axis_index("s")
        wid = c * _NS + s
        pltpu.sync_copy(dst_hbm.at[pl.ds(wid * chunks, chunks)], idx_v)

        def fill(i, _):
            for col in range(_DEGW // 16):
                ones_v[i, pl.ds(col * 16, 16)] = jnp.zeros((16,), jnp.float32)
            return 0

        lax.fori_loop(0, _C, fill, 0)
        for t in range(rows_per_tile // _C):
            pltpu.sync_copy(
                ones_v, deg_sp.at[pl.ds(s * rows_per_tile + t * _C, _C)])

        def fill1(i, _):
            for col in range(_DEGW // 16):
                ones_v[i, pl.ds(col * 16, 16)] = jnp.full((16,), 1.0,
                                                          jnp.float32)
            return 0

        lax.fori_loop(0, _C, fill1, 0)
        plsc.subcore_barrier()

        def chunk_body(j, _):
            pltpu.sync_copy(ones_v, deg_sp.at[idx_v.at[j]], add=True)
            return 0

        lax.fori_loop(0, chunks, chunk_body, 0)
        plsc.subcore_barrier()
        pltpu.sync_copy(
            deg_sp.at[pl.ds(s * rows_per_tile, rows_per_tile)],
            out_hbm.at[pl.ds(c * acc_rows + s * rows_per_tile, rows_per_tile)])

    return k(dst2)


def _sc_scatter_rows(src2d0, src2d1, dst2, xws2, *, chunks_sc, acc_rows,
                     feat):
    half = feat // 2
    rows_per_tile = acc_rows // _NS
    mesh = plsc.VectorSubcoreMesh(core_axis_name="c", subcore_axis_name="s")

    @functools.partial(
        pl.kernel,
        out_type=jax.ShapeDtypeStruct((_NC * acc_rows, half), jnp.float32),
        mesh=mesh,
        scratch_types=[
            pltpu.VMEM((_IB, _C), jnp.int32),
            pltpu.VMEM((_IB, _C), jnp.int32),
            pltpu.VMEM((_C, half), jnp.float32),
            pltpu.VMEM_SHARED((acc_rows, half), jnp.float32),
            pltpu.SemaphoreType.DMA,
        ],
    )
    def k(s0_hbm, s1_hbm, dst_hbm, xws_hbm, out_hbm, sidx_v, didx_v, rb0,
          acc_sp, gsem):
        c = lax.axis_index("c")
        s = lax.axis_index("s")

        def fillz(i, _):
            for col in range(half // 16):
                rb0[i, pl.ds(col * 16, 16)] = jnp.zeros((16,), jnp.float32)
            return 0

        lax.fori_loop(0, _C, fillz, 0)
        for t in range(rows_per_tile // _C):
            pltpu.sync_copy(
                rb0, acc_sp.at[pl.ds(s * rows_per_tile + t * _C, _C)])
        plsc.subcore_barrier()

        def group_body(gi, _):
            base = s * chunks_sc + gi * _IB

            @pl.when(c == 0)
            def _():
                pltpu.sync_copy(s0_hbm.at[pl.ds(base, _IB)], sidx_v)

            @pl.when(c != 0)
            def _():
                pltpu.sync_copy(s1_hbm.at[pl.ds(base, _IB)], sidx_v)

            pltpu.sync_copy(dst_hbm.at[pl.ds(base, _IB)], didx_v)

            def chunk_body(j, _):
                pltpu.async_copy(xws_hbm.at[sidx_v.at[j]], rb0, gsem).wait()
                pltpu.sync_copy(rb0, acc_sp.at[didx_v.at[j]], add=True)
                return 0

            lax.fori_loop(0, _IB, chunk_body, 0)
            return 0

        lax.fori_loop(0, chunks_sc // _IB, group_body, 0)
        plsc.subcore_barrier()
        pltpu.sync_copy(
            acc_sp.at[pl.ds(s * rows_per_tile, rows_per_tile)],
            out_hbm.at[pl.ds(c * acc_rows + s * rows_per_tile, rows_per_tile)])

    return k(src2d0, src2d1, dst2, xws2)


def _tc_conv_scale(degp, x, conv_W, *, n, acc_rows):
    hid = conv_W.shape[1]

    def body(degp_ref, x_ref, w_ref, xws_ref, dinv_ref):
        d0 = degp_ref[0:n, 0:1]
        d1 = degp_ref[acc_rows:acc_rows + n, 0:1]
        dinv = lax.rsqrt(d0 + d1 + 1.0)
        xw = jnp.dot(x_ref[...], w_ref[...],
                     preferred_element_type=jnp.float32)
        xws_ref[...] = xw * dinv
        dinv_ref[...] = dinv

    return pl.pallas_call(
        body,
        out_shape=[jax.ShapeDtypeStruct((n, hid), jnp.float32),
                   jax.ShapeDtypeStruct((n, 1), jnp.float32)],
    )(degp, x, conv_W)


def _tc_dense(accp, xws, dinv2, batch2, conv_b, bn1_g, bn1_b, lin1_W, lin1_b,
              bn2_g, bn2_b, lin2_W, lin2_b, bn3_g, bn3_b, *, n, acc_rows, g):
    def body(accp_ref, xws_ref, dinv_ref, batch_ref, conv_b_ref, bn1_g_ref,
             bn1_b_ref, lin1_W_ref, lin1_b_ref, bn2_g_ref, bn2_b_ref,
             lin2_W_ref, lin2_b_ref, bn3_g_ref, bn3_b_ref, out_ref):
        acc = jnp.concatenate(
            [accp_ref[0:n, :], accp_ref[acc_rows:acc_rows + n, :]], axis=1)
        dinv = dinv_ref[...]
        h = dinv * (acc + xws_ref[...]) + conv_b_ref[...]
        m = jnp.mean(h, axis=0, keepdims=True)
        v = jnp.mean((h - m) ** 2, axis=0, keepdims=True)
        hn = (h - m) * lax.rsqrt(v + 1e-5) * bn1_g_ref[...] + bn1_b_ref[...]
        hn = jnp.maximum(hn, 0.0)
        onehot = (batch_ref[...] == lax.broadcasted_iota(
            jnp.int32, (n, g), 1)).astype(jnp.float32)
        psum = lax.dot_general(onehot, hn, (((0,), (0,)), ((), ())),
                               preferred_element_type=jnp.float32,
                               precision=lax.Precision.HIGHEST)
        cnt = lax.dot_general(onehot, jnp.ones((n, 1), jnp.float32),
                              (((0,), (0,)), ((), ())),
                              preferred_element_type=jnp.float32,
                              precision=lax.Precision.HIGHEST)
        pooled = psum / jnp.maximum(cnt, 1.0)
        z = jnp.dot(pooled, lin1_W_ref[...],
                    preferred_element_type=jnp.float32) + lin1_b_ref[...]
        m2 = jnp.mean(z, axis=0, keepdims=True)
        v2 = jnp.mean((z - m2) ** 2, axis=0, keepdims=True)
        z = (z - m2) * lax.rsqrt(v2 + 1e-5) * bn2_g_ref[...] + bn2_b_ref[...]
        z = jnp.maximum(z, 0.0)
        z2 = jnp.dot(z, lin2_W_ref[...],
                     preferred_element_type=jnp.float32) + lin2_b_ref[...]
        m3 = jnp.mean(z2, axis=0, keepdims=True)
        v3 = jnp.mean((z2 - m3) ** 2, axis=0, keepdims=True)
        out_ref[...] = ((z2 - m3) * lax.rsqrt(v3 + 1e-5) * bn3_g_ref[...]
                        + bn3_b_ref[...])

    return pl.pallas_call(
        body,
        out_shape=jax.ShapeDtypeStruct((g, lin2_W.shape[1]), jnp.float32),
    )(accp, xws, dinv2, batch2, conv_b.reshape(1, -1), bn1_g.reshape(1, -1),
      bn1_b.reshape(1, -1), lin1_W, lin1_b.reshape(1, -1),
      bn2_g.reshape(1, -1), bn2_b.reshape(1, -1), lin2_W,
      lin2_b.reshape(1, -1), bn3_g.reshape(1, -1), bn3_b.reshape(1, -1))


def kernel(x, edge_index, batch, conv_W, conv_b, bn1_g, bn1_b, lin1_W,
           lin1_b, bn2_g, bn2_b, lin2_W, lin2_b, bn3_g, bn3_b):
    n = x.shape[0]
    hid = conv_W.shape[1]
    e = edge_index.shape[1]
    g = 64

    chunks = -(-(-(-e // (_NW * _C))) // 8) * 8
    ep = _NW * chunks * _C
    acc_rows = -(-(n + 1) // (_NS * _C)) * (_NS * _C)
    src = edge_index[0]
    dst = edge_index[1]
    pad = ep - e
    srcp = jnp.concatenate([src, jnp.zeros((pad,), jnp.int32)])
    dstp = jnp.concatenate([dst, jnp.full((pad,), n, jnp.int32)])
    src2d0 = (srcp * 2).reshape(_NW * chunks, _C)
    src2d1 = (srcp * 2 + 1).reshape(_NW * chunks, _C)
    dst2 = dstp.reshape(_NW * chunks, _C)

    degp = _sc_degree(dst2, chunks=chunks, acc_rows=acc_rows)
    xws, dinv2 = _tc_conv_scale(degp, x, conv_W, n=n, acc_rows=acc_rows)
    xws2 = xws.reshape(2 * n, hid // 2)
    accp = _sc_scatter_rows(src2d0, src2d1, dst2, xws2,
                            chunks_sc=(_NW * chunks) // _NS,
                            acc_rows=acc_rows, feat=hid)
    batch2 = batch.reshape(n, 1)
    return _tc_dense(accp, xws, dinv2, batch2, conv_b, bn1_g, bn1_b, lin1_W,
                     lin1_b, bn2_g, bn2_b, lin2_W, lin2_b, bn3_g, bn3_b,
                     n=n, acc_rows=acc_rows, g=g)


# speedup vs baseline: 10.5949x; 1.0390x over previous
import functools

import jax
import jax.numpy as jnp
from jax import lax
from jax.experimental import pallas as pl
from jax.experimental.pallas import tpu as pltpu
from jax.experimental.pallas import tpu_sc as plsc

_NC = 2
_NS = 16
_NW = _NC * _NS
_C = 128
_DEGW = 128
_IB = 16


def _sc_degree(dst2, *, chunks, acc_rows):
    rows_per_tile = acc_rows // _NS
    mesh = plsc.VectorSubcoreMesh(core_axis_name="c", subcore_axis_name="s")

    @functools.partial(
        pl.kernel,
        out_type=jax.ShapeDtypeStruct((_NC * acc_rows, _DEGW), jnp.float32),
        mesh=mesh,
        scratch_types=[
            pltpu.VMEM((chunks, _C), jnp.int32),
            pltpu.VMEM((_C, _DEGW), jnp.float32),
            pltpu.VMEM_SHARED((acc_rows, _DEGW), jnp.float32),
        ],
    )
    def k(dst_hbm, out_hbm, idx_v, ones_v, deg_sp):
        c = lax.axis_index("c")
        s = lax.axis_index("s")
        wid = c * _NS + s
        pltpu.sync_copy(dst_hbm.at[pl.ds(wid * chunks, chunks)], idx_v)

        def fill(i, _):
            for col in range(_DEGW // 16):
                ones_v[i, pl.ds(col * 16, 16)] = jnp.zeros((16,), jnp.float32)
            return 0

        lax.fori_loop(0, _C, fill, 0)
        for t in range(rows_per_tile // _C):
            pltpu.sync_copy(
                ones_v, deg_sp.at[pl.ds(s * rows_per_tile + t * _C, _C)])

        def fill1(i, _):
            for col in range(_DEGW // 16):
                ones_v[i, pl.ds(col * 16, 16)] = jnp.full((16,), 1.0,
                                                          jnp.float32)
            return 0

        lax.fori_loop(0, _C, fill1, 0)
        plsc.subcore_barrier()

        def chunk_body(j, _):
            pltpu.sync_copy(ones_v, deg_sp.at[idx_v.at[j]], add=True)
            return 0

        lax.fori_loop(0, chunks, chunk_body, 0)
        plsc.subcore_barrier()
        pltpu.sync_copy(
            deg_sp.at[pl.ds(s * rows_per_tile, rows_per_tile)],
            out_hbm.at[pl.ds(c * acc_rows + s * rows_per_tile, rows_per_tile)])

    return k(dst2)


def _sc_scatter_rows(src2d0, src2d1, dst2, xws2, *, chunks_sc, acc_rows,
                     feat):
    half = feat // 2
    rows_per_tile = acc_rows // _NS
    mesh = plsc.VectorSubcoreMesh(core_axis_name="c", subcore_axis_name="s")

    @functools.partial(
        pl.kernel,
        out_type=jax.ShapeDtypeStruct((_NC * acc_rows, half), jnp.float32),
        mesh=mesh,
        scratch_types=[
            pltpu.VMEM((_IB, _C), jnp.int32),
            pltpu.VMEM((_IB, _C), jnp.int32),
            pltpu.VMEM((_C, half), jnp.float32),
            pltpu.VMEM((_C, half), jnp.float32),
            pltpu.VMEM_SHARED((acc_rows, half), jnp.float32),
            pltpu.SemaphoreType.DMA,
            pltpu.SemaphoreType.DMA,
        ],
    )
    def k(s0_hbm, s1_hbm, dst_hbm, xws_hbm, out_hbm, sidx_v, didx_v, rb0,
          rb1, acc_sp, gsem0, gsem1):
        c = lax.axis_index("c")
        s = lax.axis_index("s")

        def fillz(i, _):
            for col in range(half // 16):
                rb0[i, pl.ds(col * 16, 16)] = jnp.zeros((16,), jnp.float32)
            return 0

        lax.fori_loop(0, _C, fillz, 0)
        for t in range(rows_per_tile // _C):
            pltpu.sync_copy(
                rb0, acc_sp.at[pl.ds(s * rows_per_tile + t * _C, _C)])
        plsc.subcore_barrier()

        def group_body(gi, _):
            base = s * chunks_sc + gi * _IB

            @pl.when(c == 0)
            def _():
                pltpu.sync_copy(s0_hbm.at[pl.ds(base, _IB)], sidx_v)

            @pl.when(c != 0)
            def _():
                pltpu.sync_copy(s1_hbm.at[pl.ds(base, _IB)], sidx_v)

            pltpu.sync_copy(dst_hbm.at[pl.ds(base, _IB)], didx_v)

            def pair_body(p, _):
                j0 = 2 * p
                j1 = 2 * p + 1
                d0 = pltpu.async_copy(xws_hbm.at[sidx_v.at[j0]], rb0, gsem0)
                d1 = pltpu.async_copy(xws_hbm.at[sidx_v.at[j1]], rb1, gsem1)
                d0.wait()
                pltpu.sync_copy(rb0, acc_sp.at[didx_v.at[j0]], add=True)
                d1.wait()
                pltpu.sync_copy(rb1, acc_sp.at[didx_v.at[j1]], add=True)
                return 0

            lax.fori_loop(0, _IB // 2, pair_body, 0)
            return 0

        lax.fori_loop(0, chunks_sc // _IB, group_body, 0)
        plsc.subcore_barrier()
        pltpu.sync_copy(
            acc_sp.at[pl.ds(s * rows_per_tile, rows_per_tile)],
            out_hbm.at[pl.ds(c * acc_rows + s * rows_per_tile, rows_per_tile)])

    return k(src2d0, src2d1, dst2, xws2)


def _tc_conv_scale(degp, x, conv_W, *, n, acc_rows):
    hid = conv_W.shape[1]

    def body(degp_ref, x_ref, w_ref, xws_ref, dinv_ref):
        d0 = degp_ref[0:n, 0:1]
        d1 = degp_ref[acc_rows:acc_rows + n, 0:1]
        dinv = lax.rsqrt(d0 + d1 + 1.0)
        xw = jnp.dot(x_ref[...], w_ref[...],
                     preferred_element_type=jnp.float32)
        xws_ref[...] = xw * dinv
        dinv_ref[...] = dinv

    return pl.pallas_call(
        body,
        out_shape=[jax.ShapeDtypeStruct((n, hid), jnp.float32),
                   jax.ShapeDtypeStruct((n, 1), jnp.float32)],
    )(degp, x, conv_W)


def _tc_dense(accp, xws, dinv2, batch2, conv_b, bn1_g, bn1_b, lin1_W, lin1_b,
              bn2_g, bn2_b, lin2_W, lin2_b, bn3_g, bn3_b, *, n, acc_rows, g):
    def body(accp_ref, xws_ref, dinv_ref, batch_ref, conv_b_ref, bn1_g_ref,
             bn1_b_ref, lin1_W_ref, lin1_b_ref, bn2_g_ref, bn2_b_ref,
             lin2_W_ref, lin2_b_ref, bn3_g_ref, bn3_b_ref, out_ref):
        acc = jnp.concatenate(
            [accp_ref[0:n, :], accp_ref[acc_rows:acc_rows + n, :]], axis=1)
        dinv = dinv_ref[...]
        h = dinv * (acc + xws_ref[...]) + conv_b_ref[...]
        m = jnp.mean(h, axis=0, keepdims=True)
        v = jnp.mean((h - m) ** 2, axis=0, keepdims=True)
        hn = (h - m) * lax.rsqrt(v + 1e-5) * bn1_g_ref[...] + bn1_b_ref[...]
        hn = jnp.maximum(hn, 0.0)
        onehot = (batch_ref[...] == lax.broadcasted_iota(
            jnp.int32, (n, g), 1)).astype(jnp.float32)
        psum = lax.dot_general(onehot, hn, (((0,), (0,)), ((), ())),
                               preferred_element_type=jnp.float32,
                               precision=lax.Precision.HIGHEST)
        cnt = lax.dot_general(onehot, jnp.ones((n, 1), jnp.float32),
                              (((0,), (0,)), ((), ())),
                              preferred_element_type=jnp.float32,
                              precision=lax.Precision.HIGHEST)
        pooled = psum / jnp.maximum(cnt, 1.0)
        z = jnp.dot(pooled, lin1_W_ref[...],
                    preferred_element_type=jnp.float32) + lin1_b_ref[...]
        m2 = jnp.mean(z, axis=0, keepdims=True)
        v2 = jnp.mean((z - m2) ** 2, axis=0, keepdims=True)
        z = (z - m2) * lax.rsqrt(v2 + 1e-5) * bn2_g_ref[...] + bn2_b_ref[...]
        z = jnp.maximum(z, 0.0)
        z2 = jnp.dot(z, lin2_W_ref[...],
                     preferred_element_type=jnp.float32) + lin2_b_ref[...]
        m3 = jnp.mean(z2, axis=0, keepdims=True)
        v3 = jnp.mean((z2 - m3) ** 2, axis=0, keepdims=True)
        out_ref[...] = ((z2 - m3) * lax.rsqrt(v3 + 1e-5) * bn3_g_ref[...]
                        + bn3_b_ref[...])

    return pl.pallas_call(
        body,
        out_shape=jax.ShapeDtypeStruct((g, lin2_W.shape[1]), jnp.float32),
    )(accp, xws, dinv2, batch2, conv_b.reshape(1, -1), bn1_g.reshape(1, -1),
      bn1_b.reshape(1, -1), lin1_W, lin1_b.reshape(1, -1),
      bn2_g.reshape(1, -1), bn2_b.reshape(1, -1), lin2_W,
      lin2_b.reshape(1, -1), bn3_g.reshape(1, -1), bn3_b.reshape(1, -1))


def kernel(x, edge_index, batch, conv_W, conv_b, bn1_g, bn1_b, lin1_W,
           lin1_b, bn2_g, bn2_b, lin2_W, lin2_b, bn3_g, bn3_b):
    n = x.shape[0]
    hid = conv_W.shape[1]
    e = edge_index.shape[1]
    g = 64

    chunks = -(-(-(-e // (_NW * _C))) // 8) * 8
    ep = _NW * chunks * _C
    acc_rows = -(-(n + 1) // (_NS * _C)) * (_NS * _C)
    src = edge_index[0]
    dst = edge_index[1]
    pad = ep - e
    srcp = jnp.concatenate([src, jnp.zeros((pad,), jnp.int32)])
    dstp = jnp.concatenate([dst, jnp.full((pad,), n, jnp.int32)])
    src2d0 = (srcp * 2).reshape(_NW * chunks, _C)
    src2d1 = (srcp * 2 + 1).reshape(_NW * chunks, _C)
    dst2 = dstp.reshape(_NW * chunks, _C)

    degp = _sc_degree(dst2, chunks=chunks, acc_rows=acc_rows)
    xws, dinv2 = _tc_conv_scale(degp, x, conv_W, n=n, acc_rows=acc_rows)
    xws2 = xws.reshape(2 * n, hid // 2)
    accp = _sc_scatter_rows(src2d0, src2d1, dst2, xws2,
                            chunks_sc=(_NW * chunks) // _NS,
                            acc_rows=acc_rows, feat=hid)
    batch2 = batch.reshape(n, 1)
    return _tc_dense(accp, xws, dinv2, batch2, conv_b, bn1_g, bn1_b, lin1_W,
                     lin1_b, bn2_g, bn2_b, lin2_W, lin2_b, bn3_g, bn3_b,
                     n=n, acc_rows=acc_rows, g=g)


# speedup vs baseline: 10.6696x; 1.0070x over previous
import functools

import jax
import jax.numpy as jnp
from jax import lax
from jax.experimental import pallas as pl
from jax.experimental.pallas import tpu as pltpu
from jax.experimental.pallas import tpu_sc as plsc

_NC = 2
_NS = 16
_NW = _NC * _NS
_C = 128
_DEGW = 128
_IB = 16


def _sc_degree(dst2, *, chunks, acc_rows):
    rows_per_tile = acc_rows // _NS
    mesh = plsc.VectorSubcoreMesh(core_axis_name="c", subcore_axis_name="s")

    @functools.partial(
        pl.kernel,
        out_type=jax.ShapeDtypeStruct((_NC * acc_rows, _DEGW), jnp.float32),
        mesh=mesh,
        scratch_types=[
            pltpu.VMEM((chunks, _C), jnp.int32),
            pltpu.VMEM((_C, _DEGW), jnp.float32),
            pltpu.VMEM_SHARED((acc_rows, _DEGW), jnp.float32),
        ],
    )
    def k(dst_hbm, out_hbm, idx_v, ones_v, deg_sp):
        c = lax.axis_index("c")
        s = lax.axis_index("s")
        wid = c * _NS + s
        pltpu.sync_copy(dst_hbm.at[pl.ds(wid * chunks, chunks)], idx_v)

        def fill(i, _):
            for col in range(_DEGW // 16):
                ones_v[i, pl.ds(col * 16, 16)] = jnp.zeros((16,), jnp.float32)
            return 0

        lax.fori_loop(0, _C, fill, 0)
        for t in range(rows_per_tile // _C):
            pltpu.sync_copy(
                ones_v, deg_sp.at[pl.ds(s * rows_per_tile + t * _C, _C)])

        def fill1(i, _):
            for col in range(_DEGW // 16):
                ones_v[i, pl.ds(col * 16, 16)] = jnp.full((16,), 1.0,
                                                          jnp.float32)
            return 0

        lax.fori_loop(0, _C, fill1, 0)
        plsc.subcore_barrier()

        def chunk_body(j, _):
            pltpu.sync_copy(ones_v, deg_sp.at[idx_v.at[j]], add=True)
            return 0

        lax.fori_loop(0, chunks, chunk_body, 0)
        plsc.subcore_barrier()
        pltpu.sync_copy(
            deg_sp.at[pl.ds(s * rows_per_tile, rows_per_tile)],
            out_hbm.at[pl.ds(c * acc_rows + s * rows_per_tile, rows_per_tile)])

    return k(dst2)


def _sc_scatter_rows(src2d0, src2d1, dst2, xws2, *, chunks_sc, acc_rows,
                     feat):
    half = feat // 2
    rows_per_tile = acc_rows // _NS
    mesh = plsc.VectorSubcoreMesh(core_axis_name="c", subcore_axis_name="s")

    @functools.partial(
        pl.kernel,
        out_type=jax.ShapeDtypeStruct((_NC * acc_rows, half), jnp.float32),
        mesh=mesh,
        scratch_types=[
            pltpu.VMEM((_IB, _C), jnp.int32),
            pltpu.VMEM((_IB, _C), jnp.int32),
            pltpu.VMEM((_C, half), jnp.float32),
            pltpu.VMEM((_C, half), jnp.float32),
            pltpu.VMEM_SHARED((acc_rows, half), jnp.float32),
            pltpu.SemaphoreType.DMA,
            pltpu.SemaphoreType.DMA,
            pltpu.SemaphoreType.DMA,
            pltpu.SemaphoreType.DMA,
        ],
    )
    def k(s0_hbm, s1_hbm, dst_hbm, xws_hbm, out_hbm, sidx_v, didx_v, rb0,
          rb1, acc_sp, gsem0, gsem1, ssem0, ssem1):
        c = lax.axis_index("c")
        s = lax.axis_index("s")

        def fillz(i, _):
            for col in range(half // 16):
                rb0[i, pl.ds(col * 16, 16)] = jnp.zeros((16,), jnp.float32)
            return 0

        lax.fori_loop(0, _C, fillz, 0)
        for t in range(rows_per_tile // _C):
            pltpu.sync_copy(
                rb0, acc_sp.at[pl.ds(s * rows_per_tile + t * _C, _C)])
        plsc.subcore_barrier()

        pairs_per_blk = _IB // 2

        def drain(rb, sem):
            pltpu.make_async_copy(rb, acc_sp.at[didx_v.at[0]], sem).wait()

        def pair_body(p, _):
            @pl.when(jnp.logical_and(p % pairs_per_blk == 0, p > 0))
            def _():
                drain(rb0, ssem0)
                drain(rb1, ssem1)

            @pl.when(p % pairs_per_blk == 0)
            def _():
                base = s * chunks_sc + (p // pairs_per_blk) * _IB

                @pl.when(c == 0)
                def _():
                    pltpu.sync_copy(s0_hbm.at[pl.ds(base, _IB)], sidx_v)

                @pl.when(c != 0)
                def _():
                    pltpu.sync_copy(s1_hbm.at[pl.ds(base, _IB)], sidx_v)

                pltpu.sync_copy(dst_hbm.at[pl.ds(base, _IB)], didx_v)

            @pl.when(jnp.logical_and(p % pairs_per_blk != 0, p > 0))
            def _():
                drain(rb0, ssem0)
                drain(rb1, ssem1)

            j0 = (2 * p) % _IB
            j1 = (2 * p + 1) % _IB
            d0 = pltpu.async_copy(xws_hbm.at[sidx_v.at[j0]], rb0, gsem0)
            d1 = pltpu.async_copy(xws_hbm.at[sidx_v.at[j1]], rb1, gsem1)
            d0.wait()
            pltpu.async_copy(rb0, acc_sp.at[didx_v.at[j0]], ssem0, add=True)
            d1.wait()
            pltpu.async_copy(rb1, acc_sp.at[didx_v.at[j1]], ssem1, add=True)
            return 0

        lax.fori_loop(0, chunks_sc // 2, pair_body, 0)
        drain(rb0, ssem0)
        drain(rb1, ssem1)
        plsc.subcore_barrier()
        pltpu.sync_copy(
            acc_sp.at[pl.ds(s * rows_per_tile, rows_per_tile)],
            out_hbm.at[pl.ds(c * acc_rows + s * rows_per_tile, rows_per_tile)])

    return k(src2d0, src2d1, dst2, xws2)


def _tc_conv_scale(degp, x, conv_W, *, n, acc_rows):
    hid = conv_W.shape[1]

    def body(degp_ref, x_ref, w_ref, xws_ref, dinv_ref):
        d0 = degp_ref[0:n, 0:1]
        d1 = degp_ref[acc_rows:acc_rows + n, 0:1]
        dinv = lax.rsqrt(d0 + d1 + 1.0)
        xw = jnp.dot(x_ref[...], w_ref[...],
                     preferred_element_type=jnp.float32)
        xws_ref[...] = xw * dinv
        dinv_ref[...] = dinv

    return pl.pallas_call(
        body,
        out_shape=[jax.ShapeDtypeStruct((n, hid), jnp.float32),
                   jax.ShapeDtypeStruct((n, 1), jnp.float32)],
    )(degp, x, conv_W)


def _tc_dense(accp, xws, dinv2, batch2, conv_b, bn1_g, bn1_b, lin1_W, lin1_b,
              bn2_g, bn2_b, lin2_W, lin2_b, bn3_g, bn3_b, *, n, acc_rows, g):
    def body(accp_ref, xws_ref, dinv_ref, batch_ref, conv_b_ref, bn1_g_ref,
             bn1_b_ref, lin1_W_ref, lin1_b_ref, bn2_g_ref, bn2_b_ref,
             lin2_W_ref, lin2_b_ref, bn3_g_ref, bn3_b_ref, out_ref):
        acc = jnp.concatenate(
            [accp_ref[0:n, :], accp_ref[acc_rows:acc_rows + n, :]], axis=1)
        dinv = dinv_ref[...]
        h = dinv * (acc + xws_ref[...]) + conv_b_ref[...]
        m = jnp.mean(h, axis=0, keepdims=True)
        v = jnp.mean((h - m) ** 2, axis=0, keepdims=True)
        hn = (h - m) * lax.rsqrt(v + 1e-5) * bn1_g_ref[...] + bn1_b_ref[...]
        hn = jnp.maximum(hn, 0.0)
        onehot = (batch_ref[...] == lax.broadcasted_iota(
            jnp.int32, (n, g), 1)).astype(jnp.float32)
        psum = lax.dot_general(onehot, hn, (((0,), (0,)), ((), ())),
                               preferred_element_type=jnp.float32,
                               precision=lax.Precision.HIGHEST)
        cnt = lax.dot_general(onehot, jnp.ones((n, 1), jnp.float32),
                              (((0,), (0,)), ((), ())),
                              preferred_element_type=jnp.float32,
                              precision=lax.Precision.HIGHEST)
        pooled = psum / jnp.maximum(cnt, 1.0)
        z = jnp.dot(pooled, lin1_W_ref[...],
                    preferred_element_type=jnp.float32) + lin1_b_ref[...]
        m2 = jnp.mean(z, axis=0, keepdims=True)
        v2 = jnp.mean((z - m2) ** 2, axis=0, keepdims=True)
        z = (z - m2) * lax.rsqrt(v2 + 1e-5) * bn2_g_ref[...] + bn2_b_ref[...]
        z = jnp.maximum(z, 0.0)
        z2 = jnp.dot(z, lin2_W_ref[...],
                     preferred_element_type=jnp.float32) + lin2_b_ref[...]
        m3 = jnp.mean(z2, axis=0, keepdims=True)
        v3 = jnp.mean((z2 - m3) ** 2, axis=0, keepdims=True)
        out_ref[...] = ((z2 - m3) * lax.rsqrt(v3 + 1e-5) * bn3_g_ref[...]
                        + bn3_b_ref[...])

    return pl.pallas_call(
        body,
        out_shape=jax.ShapeDtypeStruct((g, lin2_W.shape[1]), jnp.float32),
    )(accp, xws, dinv2, batch2, conv_b.reshape(1, -1), bn1_g.reshape(1, -1),
      bn1_b.reshape(1, -1), lin1_W, lin1_b.reshape(1, -1),
      bn2_g.reshape(1, -1), bn2_b.reshape(1, -1), lin2_W,
      lin2_b.reshape(1, -1), bn3_g.reshape(1, -1), bn3_b.reshape(1, -1))


def kernel(x, edge_index, batch, conv_W, conv_b, bn1_g, bn1_b, lin1_W,
           lin1_b, bn2_g, bn2_b, lin2_W, lin2_b, bn3_g, bn3_b):
    n = x.shape[0]
    hid = conv_W.shape[1]
    e = edge_index.shape[1]
    g = 64

    chunks = -(-(-(-e // (_NW * _C))) // 8) * 8
    ep = _NW * chunks * _C
    acc_rows = -(-(n + 1) // (_NS * _C)) * (_NS * _C)
    src = edge_index[0]
    dst = edge_index[1]
    pad = ep - e
    srcp = jnp.concatenate([src, jnp.zeros((pad,), jnp.int32)])
    dstp = jnp.concatenate([dst, jnp.full((pad,), n, jnp.int32)])
    src2d0 = (srcp * 2).reshape(_NW * chunks, _C)
    src2d1 = (srcp * 2 + 1).reshape(_NW * chunks, _C)
    dst2 = dstp.reshape(_NW * chunks, _C)

    degp = _sc_degree(dst2, chunks=chunks, acc_rows=acc_rows)
    xws, dinv2 = _tc_conv_scale(degp, x, conv_W, n=n, acc_rows=acc_rows)
    xws2 = xws.reshape(2 * n, hid // 2)
    accp = _sc_scatter_rows(src2d0, src2d1, dst2, xws2,
                            chunks_sc=(_NW * chunks) // _NS,
                            acc_rows=acc_rows, feat=hid)
    batch2 = batch.reshape(n, 1)
    return _tc_dense(accp, xws, dinv2, batch2, conv_b, bn1_g, bn1_b, lin1_W,
                     lin1_b, bn2_g, bn2_b, lin2_W, lin2_b, bn3_g, bn3_b,
                     n=n, acc_rows=acc_rows, g=g)


# speedup vs baseline: 10.6719x; 1.0002x over previous
import functools

import jax
import jax.numpy as jnp
from jax import lax
from jax.experimental import pallas as pl
from jax.experimental.pallas import tpu as pltpu
from jax.experimental.pallas import tpu_sc as plsc

_NC = 2
_NS = 16
_NW = _NC * _NS
_C = 128
_DEGW = 128
_IB = 16


def _sc_degree(dst2, *, chunks, acc_rows):
    rows_per_tile = acc_rows // _NS
    mesh = plsc.VectorSubcoreMesh(core_axis_name="c", subcore_axis_name="s")

    @functools.partial(
        pl.kernel,
        out_type=jax.ShapeDtypeStruct((_NC * acc_rows, _DEGW), jnp.float32),
        mesh=mesh,
        scratch_types=[
            pltpu.VMEM((chunks, _C), jnp.int32),
            pltpu.VMEM((_C, _DEGW), jnp.float32),
            pltpu.VMEM_SHARED((acc_rows, _DEGW), jnp.float32),
        ],
    )
    def k(dst_hbm, out_hbm, idx_v, ones_v, deg_sp):
        c = lax.axis_index("c")
        s = lax.axis_index("s")
        wid = c * _NS + s
        pltpu.sync_copy(dst_hbm.at[pl.ds(wid * chunks, chunks)], idx_v)

        def fill(i, _):
            for col in range(_DEGW // 16):
                ones_v[i, pl.ds(col * 16, 16)] = jnp.zeros((16,), jnp.float32)
            return 0

        lax.fori_loop(0, _C, fill, 0)
        for t in range(rows_per_tile // _C):
            pltpu.sync_copy(
                ones_v, deg_sp.at[pl.ds(s * rows_per_tile + t * _C, _C)])

        def fill1(i, _):
            for col in range(_DEGW // 16):
                ones_v[i, pl.ds(col * 16, 16)] = jnp.full((16,), 1.0,
                                                          jnp.float32)
            return 0

        lax.fori_loop(0, _C, fill1, 0)
        plsc.subcore_barrier()

        def chunk_body(j, _):
            pltpu.sync_copy(ones_v, deg_sp.at[idx_v.at[j]], add=True)
            return 0

        lax.fori_loop(0, chunks, chunk_body, 0)
        plsc.subcore_barrier()
        pltpu.sync_copy(
            deg_sp.at[pl.ds(s * rows_per_tile, rows_per_tile)],
            out_hbm.at[pl.ds(c * acc_rows + s * rows_per_tile, rows_per_tile)])

    return k(dst2)


def _sc_scatter_rows(src2d0, src2d1, dst2, xws2, *, chunks_sc, acc_rows,
                     feat):
    half = feat // 2
    rows_per_tile = acc_rows // _NS
    mesh = plsc.VectorSubcoreMesh(core_axis_name="c", subcore_axis_name="s")

    @functools.partial(
        pl.kernel,
        out_type=jax.ShapeDtypeStruct((_NC * acc_rows, half), jnp.float32),
        mesh=mesh,
        scratch_types=[
            pltpu.VMEM((_IB, _C), jnp.int32),
            pltpu.VMEM((_IB, _C), jnp.int32),
            pltpu.VMEM((_C, half), jnp.float32),
            pltpu.VMEM((_C, half), jnp.float32),
            pltpu.VMEM_SHARED((acc_rows, half), jnp.float32),
            pltpu.SemaphoreType.DMA,
            pltpu.SemaphoreType.DMA,
            pltpu.SemaphoreType.DMA,
            pltpu.SemaphoreType.DMA,
        ],
    )
    def k(s0_hbm, s1_hbm, dst_hbm, xws_hbm, out_hbm, sidx_v, didx_v, rb0,
          rb1, acc_sp, gsem0, gsem1, ssem0, ssem1):
        c = lax.axis_index("c")
        s = lax.axis_index("s")

        def fillz(i, _):
            for col in range(half // 16):
                rb0[i, pl.ds(col * 16, 16)] = jnp.zeros((16,), jnp.float32)
            return 0

        lax.fori_loop(0, _C, fillz, 0)
        for t in range(rows_per_tile // _C):
            pltpu.sync_copy(
                rb0, acc_sp.at[pl.ds(s * rows_per_tile + t * _C, _C)])
        plsc.subcore_barrier()

        pairs_per_blk = _IB // 2

        def drain(rb, sem):
            pltpu.make_async_copy(rb, acc_sp.at[didx_v.at[0]], sem).wait()

        def pair_body(p, _):
            @pl.when(jnp.logical_and(p % pairs_per_blk == 0, p > 0))
            def _():
                drain(rb0, ssem0)
                drain(rb1, ssem1)

            @pl.when(p % pairs_per_blk == 0)
            def _():
                base = s * chunks_sc + (p // pairs_per_blk) * _IB

                @pl.when(c == 0)
                def _():
                    pltpu.sync_copy(s0_hbm.at[pl.ds(base, _IB)], sidx_v)

                @pl.when(c != 0)
                def _():
                    pltpu.sync_copy(s1_hbm.at[pl.ds(base, _IB)], sidx_v)

                pltpu.sync_copy(dst_hbm.at[pl.ds(base, _IB)], didx_v)

            @pl.when(jnp.logical_and(p % pairs_per_blk != 0, p > 0))
            def _():
                drain(rb0, ssem0)
                drain(rb1, ssem1)

            j0 = (2 * p) % _IB
            j1 = (2 * p + 1) % _IB
            d0 = pltpu.async_copy(xws_hbm.at[sidx_v.at[j0]], rb0, gsem0)
            d1 = pltpu.async_copy(xws_hbm.at[sidx_v.at[j1]], rb1, gsem1)
            d0.wait()
            pltpu.async_copy(rb0, acc_sp.at[didx_v.at[j0]], ssem0, add=True)
            d1.wait()
            pltpu.async_copy(rb1, acc_sp.at[didx_v.at[j1]], ssem1, add=True)
            return 0

        lax.fori_loop(0, chunks_sc // 2, pair_body, 0)
        drain(rb0, ssem0)
        drain(rb1, ssem1)
        plsc.subcore_barrier()
        pltpu.sync_copy(
            acc_sp.at[pl.ds(s * rows_per_tile, rows_per_tile)],
            out_hbm.at[pl.ds(c * acc_rows + s * rows_per_tile, rows_per_tile)])

    return k(src2d0, src2d1, dst2, xws2)


def _tc_xw(x, conv_W):
    def body(x_ref, w_ref, xw_ref):
        xw_ref[...] = jnp.dot(x_ref[...], w_ref[...],
                              preferred_element_type=jnp.float32)

    return pl.pallas_call(
        body,
        out_shape=jax.ShapeDtypeStruct((x.shape[0], conv_W.shape[1]),
                                       jnp.float32),
    )(x, conv_W)


def _tc_scale(degp, xw, *, n, acc_rows):
    def body(degp_ref, xw_ref, xws_ref, dinv_ref):
        d0 = degp_ref[0:n, 0:1]
        d1 = degp_ref[acc_rows:acc_rows + n, 0:1]
        dinv = lax.rsqrt(d0 + d1 + 1.0)
        xws_ref[...] = xw_ref[...] * dinv
        dinv_ref[...] = dinv

    return pl.pallas_call(
        body,
        out_shape=[jax.ShapeDtypeStruct(xw.shape, jnp.float32),
                   jax.ShapeDtypeStruct((n, 1), jnp.float32)],
    )(degp, xw)


def _tc_dense(accp, xws, dinv2, batch2, conv_b, bn1_g, bn1_b, lin1_W, lin1_b,
              bn2_g, bn2_b, lin2_W, lin2_b, bn3_g, bn3_b, *, n, acc_rows, g):
    def body(accp_ref, xws_ref, dinv_ref, batch_ref, conv_b_ref, bn1_g_ref,
             bn1_b_ref, lin1_W_ref, lin1_b_ref, bn2_g_ref, bn2_b_ref,
             lin2_W_ref, lin2_b_ref, bn3_g_ref, bn3_b_ref, out_ref):
        acc = jnp.concatenate(
            [accp_ref[0:n, :], accp_ref[acc_rows:acc_rows + n, :]], axis=1)
        dinv = dinv_ref[...]
        h = dinv * (acc + xws_ref[...]) + conv_b_ref[...]
        m = jnp.mean(h, axis=0, keepdims=True)
        v = jnp.mean((h - m) ** 2, axis=0, keepdims=True)
        hn = (h - m) * lax.rsqrt(v + 1e-5) * bn1_g_ref[...] + bn1_b_ref[...]
        hn = jnp.maximum(hn, 0.0)
        onehot = (batch_ref[...] == lax.broadcasted_iota(
            jnp.int32, (n, g), 1)).astype(jnp.float32)
        psum = lax.dot_general(onehot, hn, (((0,), (0,)), ((), ())),
                               preferred_element_type=jnp.float32,
                               precision=lax.Precision.HIGHEST)
        cnt = lax.dot_general(onehot, jnp.ones((n, 1), jnp.float32),
                              (((0,), (0,)), ((), ())),
                              preferred_element_type=jnp.float32,
                              precision=lax.Precision.HIGHEST)
        pooled = psum / jnp.maximum(cnt, 1.0)
        z = jnp.dot(pooled, lin1_W_ref[...],
                    preferred_element_type=jnp.float32) + lin1_b_ref[...]
        m2 = jnp.mean(z, axis=0, keepdims=True)
        v2 = jnp.mean((z - m2) ** 2, axis=0, keepdims=True)
        z = (z - m2) * lax.rsqrt(v2 + 1e-5) * bn2_g_ref[...] + bn2_b_ref[...]
        z = jnp.maximum(z, 0.0)
        z2 = jnp.dot(z, lin2_W_ref[...],
                     preferred_element_type=jnp.float32) + lin2_b_ref[...]
        m3 = jnp.mean(z2, axis=0, keepdims=True)
        v3 = jnp.mean((z2 - m3) ** 2, axis=0, keepdims=True)
        out_ref[...] = ((z2 - m3) * lax.rsqrt(v3 + 1e-5) * bn3_g_ref[...]
                        + bn3_b_ref[...])

    return pl.pallas_call(
        body,
        out_shape=jax.ShapeDtypeStruct((g, lin2_W.shape[1]), jnp.float32),
    )(accp, xws, dinv2, batch2, conv_b.reshape(1, -1), bn1_g.reshape(1, -1),
      bn1_b.reshape(1, -1), lin1_W, lin1_b.reshape(1, -1),
      bn2_g.reshape(1, -1), bn2_b.reshape(1, -1), lin2_W,
      lin2_b.reshape(1, -1), bn3_g.reshape(1, -1), bn3_b.reshape(1, -1))


def kernel(x, edge_index, batch, conv_W, conv_b, bn1_g, bn1_b, lin1_W,
           lin1_b, bn2_g, bn2_b, lin2_W, lin2_b, bn3_g, bn3_b):
    n = x.shape[0]
    hid = conv_W.shape[1]
    e = edge_index.shape[1]
    g = 64

    chunks = -(-(-(-e // (_NW * _C))) // 8) * 8
    ep = _NW * chunks * _C
    acc_rows = -(-(n + 1) // (_NS * _C)) * (_NS * _C)
    src = edge_index[0]
    dst = edge_index[1]
    pad = ep - e
    srcp = jnp.concatenate([src, jnp.zeros((pad,), jnp.int32)])
    dstp = jnp.concatenate([dst, jnp.full((pad,), n, jnp.int32)])
    src2d0 = (srcp * 2).reshape(_NW * chunks, _C)
    src2d1 = (srcp * 2 + 1).reshape(_NW * chunks, _C)
    dst2 = dstp.reshape(_NW * chunks, _C)

    degp = _sc_degree(dst2, chunks=chunks, acc_rows=acc_rows)
    xw = _tc_xw(x, conv_W)
    xws, dinv2 = _tc_scale(degp, xw, n=n, acc_rows=acc_rows)
    xws2 = xws.reshape(2 * n, hid // 2)
    accp = _sc_scatter_rows(src2d0, src2d1, dst2, xws2,
                            chunks_sc=(_NW * chunks) // _NS,
                            acc_rows=acc_rows, feat=hid)
    batch2 = batch.reshape(n, 1)
    return _tc_dense(accp, xws, dinv2, batch2, conv_b, bn1_g, bn1_b, lin1_W,
                     lin1_b, bn2_g, bn2_b, lin2_W, lin2_b, bn3_g, bn3_b,
                     n=n, acc_rows=acc_rows, g=g)
